# Initial kernel scaffold; baseline (speedup 1.0000x reference)
import jax
import jax.numpy as jnp
from jax.experimental import pallas as pl


def kernel(hidden_states, aw1, ab1, aw2, ab2, pw1, pb1, pw2, pb2, ent):
    raise NotImplementedError("write your pallas kernel here")


# speedup vs baseline: 3.8220x; 3.8220x over previous
import math

import jax
import jax.numpy as jnp
from jax.experimental import pallas as pl
from jax.experimental.pallas import tpu as pltpu

_E = 16
_STRENGTH = 0.5
_SQRT_HALF = 0.7071067811865476


def _rot_matrix_kernel(ent_ref, r_ref):
    ang = ent_ref[...] * _STRENGTH
    c_all = jnp.cos(ang)
    s_all = jnp.sin(ang)
    rows = jax.lax.broadcasted_iota(jnp.int32, (_E, _E), 0)
    r = jnp.eye(_E, dtype=jnp.float32)
    for i in range(_E):
        for j in range(i + 1, _E):
            c = jax.lax.slice(c_all, (i, j), (i + 1, j + 1))
            s = jax.lax.slice(s_all, (i, j), (i + 1, j + 1))
            ri = jax.lax.slice(r, (i, 0), (i + 1, _E))
            rj = jax.lax.slice(r, (j, 0), (j + 1, _E))
            r = jnp.where(rows == i, c * ri - s * rj,
                          jnp.where(rows == j, s * ri + c * rj, r))
    r_ref[...] = r


def _gelu_exact(x):
    return 0.5 * x * (1.0 + jax.lax.erf(x * _SQRT_HALF))


def _dot_t(a, b):
    return jax.lax.dot_general(a, b, (((1,), (1,)), ((), ())))


def _dot_h(a, b):
    return jax.lax.dot_general(a, b, (((1,), (1,)), ((), ())),
                               precision=jax.lax.Precision.HIGHEST)


def _router_kernel(x_ref, aw1_ref, ab1_ref, aw2_ref, ab2_ref,
                   pw1_ref, pb1_ref, pw2_ref, pb2_ref, r_ref,
                   tkp_ref, tki_ref, deco_ref, rph_ref, mp_ref):
    x = x_ref[...]

    ha = _gelu_exact(_dot_t(x, aw1_ref[...]) + ab1_ref[...])
    raw_amp = _dot_t(ha, aw2_ref[...]) + ab2_ref[...]

    hp = _gelu_exact(_dot_t(x, pw1_ref[...]) + pb1_ref[...])
    raw_ph = jnp.tanh(_dot_t(hp, pw2_ref[...]) + pb2_ref[...])
    rph_ref[...] = raw_ph * math.pi

    a = jnp.abs(raw_amp)
    a = a - jnp.max(a, axis=-1, keepdims=True)
    ea = jnp.exp(a)
    amps = jnp.sqrt(ea / jnp.sum(ea, axis=-1, keepdims=True))

    deco = _dot_h(amps, r_ref[...])
    deco_ref[...] = deco

    mp = deco * deco
    mp = mp / jnp.maximum(jnp.sum(mp, axis=-1, keepdims=True), 1e-12)
    mp_ref[...] = mp

    idx = jax.lax.broadcasted_iota(jnp.int32, mp.shape, 1)
    p1 = jnp.max(mp, axis=-1, keepdims=True)
    i1 = jnp.min(jnp.where(mp == p1, idx, _E), axis=-1, keepdims=True)
    masked = jnp.where(idx == i1, -1.0, mp)
    p2 = jnp.max(masked, axis=-1, keepdims=True)
    i2 = jnp.min(jnp.where(masked == p2, idx, _E), axis=-1, keepdims=True)

    denom = jnp.maximum(p1 + p2, 1e-12)
    tkp_ref[...] = jnp.concatenate([p1, p2], axis=1) / denom
    tki_ref[...] = jnp.concatenate([i1, i2], axis=1)


def kernel(hidden_states, aw1, ab1, aw2, ab2, pw1, pb1, pw2, pb2, ent):
    b, s, h = hidden_states.shape
    n = b * s
    hh = aw1.shape[0]
    x = hidden_states.reshape(n, h)

    rot = pl.pallas_call(
        _rot_matrix_kernel,
        out_shape=jax.ShapeDtypeStruct((_E, _E), jnp.float32),
    )(ent)

    t = 512
    grid = (n // t,)

    def tok(i):
        return (i, 0)

    def rep(i):
        return (0, 0)

    out_shapes = (
        jax.ShapeDtypeStruct((n, 2), jnp.float32),
        jax.ShapeDtypeStruct((n, 2), jnp.int32),
        jax.ShapeDtypeStruct((n, _E), jnp.float32),
        jax.ShapeDtypeStruct((n, _E), jnp.float32),
        jax.ShapeDtypeStruct((n, _E), jnp.float32),
    )

    tkp, tki, deco, rph, mp = pl.pallas_call(
        _router_kernel,
        grid=grid,
        in_specs=[
            pl.BlockSpec((t, h), tok),
            pl.BlockSpec((hh, h), rep),
            pl.BlockSpec((1, hh), rep),
            pl.BlockSpec((_E, hh), rep),
            pl.BlockSpec((1, _E), rep),
            pl.BlockSpec((hh, h), rep),
            pl.BlockSpec((1, hh), rep),
            pl.BlockSpec((_E, hh), rep),
            pl.BlockSpec((1, _E), rep),
            pl.BlockSpec((_E, _E), rep),
        ],
        out_specs=(
            pl.BlockSpec((t, 2), tok),
            pl.BlockSpec((t, 2), tok),
            pl.BlockSpec((t, _E), tok),
            pl.BlockSpec((t, _E), tok),
            pl.BlockSpec((t, _E), tok),
        ),
        out_shape=out_shapes,
        compiler_params=pltpu.CompilerParams(
            dimension_semantics=("arbitrary",),
        ),
    )(x, aw1, ab1.reshape(1, hh), aw2, ab2.reshape(1, _E),
      pw1, pb1.reshape(1, hh), pw2, pb2.reshape(1, _E), rot)

    return (
        tkp.reshape(b, s, 2),
        tki.reshape(b, s, 2),
        deco.reshape(b, s, _E),
        rph.reshape(b, s, _E),
        mp.reshape(b, s, _E),
    )
